# initial kernel scaffold (unmeasured)
import jax
import jax.numpy as jnp
from jax import lax
from jax.experimental import pallas as pl
from jax.experimental.pallas import tpu as pltpu


def kernel(
    x,
):
    def body(*refs):
        pass

    out_shape = jax.ShapeDtypeStruct(..., jnp.float32)
    return pl.pallas_call(body, out_shape=out_shape)(...)


# baseline (device time: 439742 ns/iter reference)
import jax
import jax.numpy as jnp
from jax import lax
from jax.experimental import pallas as pl
from jax.experimental.pallas import tpu as pltpu

N_CHUNKS = 8


def kernel(x):
    m, n = x.shape
    assert m % N_CHUNKS == 0
    cm = m // N_CHUNKS

    xb = x.astype(jnp.bfloat16)

    def body(x_ref, out_ref, recv_ref, send_sems, recv_sems):
        i = pl.program_id(0)
        my_x = lax.axis_index("x")
        my_y = lax.axis_index("y")
        my_z = lax.axis_index("z")
        nbr = (1 - my_x, my_y, my_z)

        @pl.when(i == 0)
        def _():
            barrier_sem = pltpu.get_barrier_semaphore()
            pl.semaphore_signal(
                barrier_sem,
                inc=1,
                device_id=nbr,
                device_id_type=pl.DeviceIdType.MESH,
            )
            pl.semaphore_wait(barrier_sem, 1)

        rdma = pltpu.make_async_remote_copy(
            src_ref=x_ref,
            dst_ref=recv_ref.at[i],
            send_sem=send_sems.at[i],
            recv_sem=recv_sems.at[i],
            device_id=nbr,
            device_id_type=pl.DeviceIdType.MESH,
        )
        rdma.start()
        rdma.wait()

        out_ref[...] = x_ref[...] + recv_ref[i]

    return pl.pallas_call(
        body,
        grid=(N_CHUNKS,),
        in_specs=[
            pl.BlockSpec((cm, n), lambda i: (i, 0), memory_space=pltpu.VMEM),
        ],
        out_specs=pl.BlockSpec((cm, n), lambda i: (i, 0), memory_space=pltpu.VMEM),
        out_shape=jax.ShapeDtypeStruct((m, n), jnp.bfloat16),
        scratch_shapes=[
            pltpu.VMEM((N_CHUNKS, cm, n), jnp.bfloat16),
            pltpu.SemaphoreType.DMA((N_CHUNKS,)),
            pltpu.SemaphoreType.DMA((N_CHUNKS,)),
        ],
        compiler_params=pltpu.CompilerParams(
            collective_id=0, vmem_limit_bytes=64 * 1024 * 1024
        ),
    )(xb)


# device time: 271494 ns/iter; 1.6197x vs baseline; 1.6197x over previous
import jax
import jax.numpy as jnp
from jax import lax
from jax.experimental import pallas as pl
from jax.experimental.pallas import tpu as pltpu

C = 8


def kernel(x):
    m, n = x.shape
    half = m // 2
    cr = half // C

    xb = x.astype(jnp.bfloat16)

    def body(
        x_ref,
        out_ref,
        xh_ref,
        recv1_ref,
        recv2_ref,
        copyin_sem,
        send1_sems,
        recv1_sems,
        send2_sems,
        recv2_sems,
        out1_sem,
        out2_sem,
    ):
        my_x = lax.axis_index("x")
        my_y = lax.axis_index("y")
        my_z = lax.axis_index("z")
        xn = (1 - my_x, my_y, my_z)
        yn = (my_x, 1 - my_y, my_z)

        barrier_sem = pltpu.get_barrier_semaphore()
        for nbr in (xn, yn):
            pl.semaphore_signal(
                barrier_sem,
                inc=1,
                device_id=nbr,
                device_id_type=pl.DeviceIdType.MESH,
            )
        pl.semaphore_wait(barrier_sem, 2)

        h0 = my_y * half

        copyin = pltpu.make_async_copy(
            x_ref.at[pl.ds(h0, half), :], xh_ref, copyin_sem
        )
        copyin.start()

        def s1(c):
            return pltpu.make_async_remote_copy(
                src_ref=x_ref.at[pl.ds(h0 + c * cr, cr), :],
                dst_ref=recv1_ref.at[pl.ds(c * cr, cr), :],
                send_sem=send1_sems.at[c],
                recv_sem=recv1_sems.at[c],
                device_id=xn,
                device_id_type=pl.DeviceIdType.MESH,
            )

        def s2(c):
            return pltpu.make_async_remote_copy(
                src_ref=recv1_ref.at[pl.ds(c * cr, cr), :],
                dst_ref=recv2_ref.at[pl.ds(c * cr, cr), :],
                send_sem=send2_sems.at[c],
                recv_sem=recv2_sems.at[c],
                device_id=yn,
                device_id_type=pl.DeviceIdType.MESH,
            )

        for c in range(C):
            s1(c).start()

        copyin.wait()

        for c in range(C):
            s1(c).wait_recv()
            sl = pl.ds(c * cr, cr)
            recv1_ref[sl, :] = xh_ref[sl, :] + recv1_ref[sl, :]
            s2(c).start()

        out1 = pltpu.make_async_copy(
            recv1_ref, out_ref.at[pl.ds(h0, half), :], out1_sem
        )
        out1.start()

        for c in range(C):
            s2(c).wait_recv()
        out2 = pltpu.make_async_copy(
            recv2_ref, out_ref.at[pl.ds((1 - my_y) * half, half), :], out2_sem
        )
        out2.start()

        for c in range(C):
            s1(c).wait_send()
            s2(c).wait_send()
        out1.wait()
        out2.wait()

    return pl.pallas_call(
        body,
        in_specs=[pl.BlockSpec(memory_space=pl.ANY)],
        out_specs=pl.BlockSpec(memory_space=pl.ANY),
        out_shape=jax.ShapeDtypeStruct((m, n), jnp.bfloat16),
        scratch_shapes=[
            pltpu.VMEM((half, n), jnp.bfloat16),
            pltpu.VMEM((half, n), jnp.bfloat16),
            pltpu.VMEM((half, n), jnp.bfloat16),
            pltpu.SemaphoreType.DMA,
            pltpu.SemaphoreType.DMA((C,)),
            pltpu.SemaphoreType.DMA((C,)),
            pltpu.SemaphoreType.DMA((C,)),
            pltpu.SemaphoreType.DMA((C,)),
            pltpu.SemaphoreType.DMA,
            pltpu.SemaphoreType.DMA,
        ],
        compiler_params=pltpu.CompilerParams(
            collective_id=0, vmem_limit_bytes=80 * 1024 * 1024
        ),
    )(xb)


# device time: 234750 ns/iter; 1.8732x vs baseline; 1.1565x over previous
import jax
import jax.numpy as jnp
from jax import lax
from jax.experimental import pallas as pl
from jax.experimental.pallas import tpu as pltpu

C = 8


def kernel(x):
    m, n = x.shape
    half = m // 2
    cr = half // C

    def body(
        x_ref,
        out_ref,
        stage_ref,
        xh_ref,
        recv1_ref,
        recv2_ref,
        in_sems,
        send1_sems,
        recv1_sems,
        send2_sems,
        recv2_sems,
        out1_sems,
        out2_sems,
    ):
        my_x = lax.axis_index("x")
        my_y = lax.axis_index("y")
        my_z = lax.axis_index("z")
        xn = (1 - my_x, my_y, my_z)
        yn = (my_x, 1 - my_y, my_z)

        barrier_sem = pltpu.get_barrier_semaphore()
        for nbr in (xn, yn):
            pl.semaphore_signal(
                barrier_sem,
                inc=1,
                device_id=nbr,
                device_id_type=pl.DeviceIdType.MESH,
            )
        pl.semaphore_wait(barrier_sem, 2)

        h0 = my_y * half

        def in_dma(c):
            return pltpu.make_async_copy(
                x_ref.at[pl.ds(h0 + c * cr, cr), :],
                stage_ref.at[c % 2],
                in_sems.at[c],
            )

        def s1(c):
            return pltpu.make_async_remote_copy(
                src_ref=xh_ref.at[pl.ds(c * cr, cr), :],
                dst_ref=recv1_ref.at[pl.ds(c * cr, cr), :],
                send_sem=send1_sems.at[c],
                recv_sem=recv1_sems.at[c],
                device_id=xn,
                device_id_type=pl.DeviceIdType.MESH,
            )

        def s2(c):
            return pltpu.make_async_remote_copy(
                src_ref=recv1_ref.at[pl.ds(c * cr, cr), :],
                dst_ref=recv2_ref.at[pl.ds(c * cr, cr), :],
                send_sem=send2_sems.at[c],
                recv_sem=recv2_sems.at[c],
                device_id=yn,
                device_id_type=pl.DeviceIdType.MESH,
            )

        in_dma(0).start()
        if C > 1:
            in_dma(1).start()
        for c in range(C):
            in_dma(c).wait()
            sl = pl.ds(c * cr, cr)
            xh_ref[sl, :] = stage_ref[c % 2].astype(jnp.bfloat16)
            if c + 2 < C:
                in_dma(c + 2).start()
            s1(c).start()

        for c in range(C):
            s1(c).wait_recv()
            sl = pl.ds(c * cr, cr)
            recv1_ref[sl, :] = xh_ref[sl, :] + recv1_ref[sl, :]
            s2(c).start()
            pltpu.make_async_copy(
                recv1_ref.at[sl, :],
                out_ref.at[pl.ds(h0 + c * cr, cr), :],
                out1_sems.at[c],
            ).start()

        oh0 = (1 - my_y) * half
        for c in range(C):
            s2(c).wait_recv()
            sl = pl.ds(c * cr, cr)
            pltpu.make_async_copy(
                recv2_ref.at[sl, :],
                out_ref.at[pl.ds(oh0 + c * cr, cr), :],
                out2_sems.at[c],
            ).start()

        for c in range(C):
            s1(c).wait_send()
            s2(c).wait_send()
            sl = pl.ds(c * cr, cr)
            pltpu.make_async_copy(
                recv1_ref.at[sl, :],
                out_ref.at[pl.ds(h0 + c * cr, cr), :],
                out1_sems.at[c],
            ).wait()
            pltpu.make_async_copy(
                recv2_ref.at[sl, :],
                out_ref.at[pl.ds(oh0 + c * cr, cr), :],
                out2_sems.at[c],
            ).wait()

    return pl.pallas_call(
        body,
        in_specs=[pl.BlockSpec(memory_space=pl.ANY)],
        out_specs=pl.BlockSpec(memory_space=pl.ANY),
        out_shape=jax.ShapeDtypeStruct((m, n), jnp.bfloat16),
        scratch_shapes=[
            pltpu.VMEM((2, cr, n), jnp.float32),
            pltpu.VMEM((half, n), jnp.bfloat16),
            pltpu.VMEM((half, n), jnp.bfloat16),
            pltpu.VMEM((half, n), jnp.bfloat16),
            pltpu.SemaphoreType.DMA((C,)),
            pltpu.SemaphoreType.DMA((C,)),
            pltpu.SemaphoreType.DMA((C,)),
            pltpu.SemaphoreType.DMA((C,)),
            pltpu.SemaphoreType.DMA((C,)),
            pltpu.SemaphoreType.DMA((C,)),
            pltpu.SemaphoreType.DMA((C,)),
        ],
        compiler_params=pltpu.CompilerParams(
            collective_id=0, vmem_limit_bytes=62 * 1024 * 1024
        ),
    )(x)


# device time: 223652 ns/iter; 1.9662x vs baseline; 1.0496x over previous
import jax
import jax.numpy as jnp
from jax import lax
from jax.experimental import pallas as pl
from jax.experimental.pallas import tpu as pltpu

C = 16


def kernel(x):
    m, n = x.shape
    half = m // 2
    cr = half // C

    def body(
        x_ref,
        out_ref,
        stage_ref,
        xh_ref,
        recv1_ref,
        recv2_ref,
        in_sems,
        send1_sems,
        recv1_sems,
        send2_sems,
        recv2_sems,
        out1_sems,
        out2_sems,
    ):
        my_x = lax.axis_index("x")
        my_y = lax.axis_index("y")
        my_z = lax.axis_index("z")
        xn = (1 - my_x, my_y, my_z)
        yn = (my_x, 1 - my_y, my_z)

        barrier_sem = pltpu.get_barrier_semaphore()
        for nbr in (xn, yn):
            pl.semaphore_signal(
                barrier_sem,
                inc=1,
                device_id=nbr,
                device_id_type=pl.DeviceIdType.MESH,
            )
        pl.semaphore_wait(barrier_sem, 2)

        h0 = my_y * half

        def in_dma(c):
            return pltpu.make_async_copy(
                x_ref.at[pl.ds(h0 + c * cr, cr), :],
                stage_ref.at[c % 2],
                in_sems.at[c],
            )

        def s1(c):
            return pltpu.make_async_remote_copy(
                src_ref=xh_ref.at[pl.ds(c * cr, cr), :],
                dst_ref=recv1_ref.at[pl.ds(c * cr, cr), :],
                send_sem=send1_sems.at[c],
                recv_sem=recv1_sems.at[c],
                device_id=xn,
                device_id_type=pl.DeviceIdType.MESH,
            )

        def s2(c):
            return pltpu.make_async_remote_copy(
                src_ref=recv1_ref.at[pl.ds(c * cr, cr), :],
                dst_ref=recv2_ref.at[pl.ds(c * cr, cr), :],
                send_sem=send2_sems.at[c],
                recv_sem=recv2_sems.at[c],
                device_id=yn,
                device_id_type=pl.DeviceIdType.MESH,
            )

        in_dma(0).start()
        if C > 1:
            in_dma(1).start()
        for c in range(C):
            in_dma(c).wait()
            sl = pl.ds(c * cr, cr)
            xh_ref[sl, :] = stage_ref[c % 2].astype(jnp.bfloat16)
            if c + 2 < C:
                in_dma(c + 2).start()
            s1(c).start()

        for c in range(C):
            s1(c).wait_recv()
            sl = pl.ds(c * cr, cr)
            recv1_ref[sl, :] = xh_ref[sl, :] + recv1_ref[sl, :]
            s2(c).start()
            pltpu.make_async_copy(
                recv1_ref.at[sl, :],
                out_ref.at[pl.ds(h0 + c * cr, cr), :],
                out1_sems.at[c],
            ).start()

        oh0 = (1 - my_y) * half
        for c in range(C):
            s2(c).wait_recv()
            sl = pl.ds(c * cr, cr)
            pltpu.make_async_copy(
                recv2_ref.at[sl, :],
                out_ref.at[pl.ds(oh0 + c * cr, cr), :],
                out2_sems.at[c],
            ).start()

        for c in range(C):
            s1(c).wait_send()
            s2(c).wait_send()
            sl = pl.ds(c * cr, cr)
            pltpu.make_async_copy(
                recv1_ref.at[sl, :],
                out_ref.at[pl.ds(h0 + c * cr, cr), :],
                out1_sems.at[c],
            ).wait()
            pltpu.make_async_copy(
                recv2_ref.at[sl, :],
                out_ref.at[pl.ds(oh0 + c * cr, cr), :],
                out2_sems.at[c],
            ).wait()

    return pl.pallas_call(
        body,
        in_specs=[pl.BlockSpec(memory_space=pl.ANY)],
        out_specs=pl.BlockSpec(memory_space=pl.ANY),
        out_shape=jax.ShapeDtypeStruct((m, n), jnp.bfloat16),
        scratch_shapes=[
            pltpu.VMEM((2, cr, n), jnp.float32),
            pltpu.VMEM((half, n), jnp.bfloat16),
            pltpu.VMEM((half, n), jnp.bfloat16),
            pltpu.VMEM((half, n), jnp.bfloat16),
            pltpu.SemaphoreType.DMA((C,)),
            pltpu.SemaphoreType.DMA((C,)),
            pltpu.SemaphoreType.DMA((C,)),
            pltpu.SemaphoreType.DMA((C,)),
            pltpu.SemaphoreType.DMA((C,)),
            pltpu.SemaphoreType.DMA((C,)),
            pltpu.SemaphoreType.DMA((C,)),
        ],
        compiler_params=pltpu.CompilerParams(
            collective_id=0, vmem_limit_bytes=62 * 1024 * 1024
        ),
    )(x)


# device time: 180181 ns/iter; 2.4406x vs baseline; 1.2413x over previous
import jax
import jax.numpy as jnp
from jax import lax
from jax.experimental import pallas as pl
from jax.experimental.pallas import tpu as pltpu

C = 8
H = C // 2


def kernel(x):
    m, n = x.shape
    Q = m // 4
    cr = Q // C

    def body(
        x_ref,
        out_ref,
        stage_ref,
        xq_ref,
        recv1_ref,
        recv2y_ref,
        recv2z_ref,
        recv3_ref,
        in_sems,
        s1_send,
        s1_recv,
        s2y_send,
        s2y_recv,
        s2z_send,
        s2z_recv,
        s3y_send,
        s3y_recv,
        s3z_send,
        s3z_recv,
        o1_sems,
        o2y_sems,
        o2z_sems,
        o3_sems,
    ):
        my_x = lax.axis_index("x")
        my_y = lax.axis_index("y")
        my_z = lax.axis_index("z")
        xn = (1 - my_x, my_y, my_z)
        yn = (my_x, 1 - my_y, my_z)
        zn = (my_x, my_y, 1 - my_z)

        barrier_sem = pltpu.get_barrier_semaphore()
        for nbr in (xn, yn, zn):
            pl.semaphore_signal(
                barrier_sem,
                inc=1,
                device_id=nbr,
                device_id_type=pl.DeviceIdType.MESH,
            )
        pl.semaphore_wait(barrier_sem, 3)

        q0 = (2 * my_y + my_z) * Q
        qy0 = (2 * (1 - my_y) + my_z) * Q
        qz0 = (2 * my_y + (1 - my_z)) * Q
        qd0 = (2 * (1 - my_y) + (1 - my_z)) * Q

        def in_dma(c):
            return pltpu.make_async_copy(
                x_ref.at[pl.ds(q0 + c * cr, cr), :],
                stage_ref.at[c % 2],
                in_sems.at[c],
            )

        def rdma(src, dst, ssem, rsem, dev):
            return pltpu.make_async_remote_copy(
                src_ref=src,
                dst_ref=dst,
                send_sem=ssem,
                recv_sem=rsem,
                device_id=dev,
                device_id_type=pl.DeviceIdType.MESH,
            )

        def chunk(ref, c):
            return ref.at[pl.ds(c * cr, cr), :]

        def s1(c):
            return rdma(
                chunk(xq_ref, c), chunk(recv1_ref, c),
                s1_send.at[c], s1_recv.at[c], xn,
            )

        def s2y(c):
            return rdma(
                chunk(recv1_ref, c), chunk(recv2y_ref, c),
                s2y_send.at[c], s2y_recv.at[c], yn,
            )

        def s2z(c):
            return rdma(
                chunk(recv1_ref, c), chunk(recv2z_ref, c),
                s2z_send.at[c], s2z_recv.at[c], zn,
            )

        def s3y(j):
            return rdma(
                chunk(recv2z_ref, j), chunk(recv3_ref, j),
                s3y_send.at[j], s3y_recv.at[j], yn,
            )

        def s3z(j):
            return rdma(
                chunk(recv2y_ref, H + j), chunk(recv3_ref, H + j),
                s3z_send.at[j], s3z_recv.at[j], zn,
            )

        def out_dma(src_ref, c, row0, sem):
            return pltpu.make_async_copy(
                chunk(src_ref, c),
                out_ref.at[pl.ds(row0 + c * cr, cr), :],
                sem,
            )

        in_dma(0).start()
        if C > 1:
            in_dma(1).start()
        for c in range(C):
            in_dma(c).wait()
            xq_ref[pl.ds(c * cr, cr), :] = stage_ref[c % 2].astype(jnp.bfloat16)
            if c + 2 < C:
                in_dma(c + 2).start()
            s1(c).start()

        for c in range(C):
            s1(c).wait_recv()
            sl = pl.ds(c * cr, cr)
            recv1_ref[sl, :] = xq_ref[sl, :] + recv1_ref[sl, :]
            s2y(c).start()
            s2z(c).start()
            out_dma(recv1_ref, c, q0, o1_sems.at[c]).start()

        for c in range(C):
            s2y(c).wait_recv()
            out_dma(recv2y_ref, c, qy0, o2y_sems.at[c]).start()
            if c >= H:
                s3z(c - H).start()
            s2z(c).wait_recv()
            out_dma(recv2z_ref, c, qz0, o2z_sems.at[c]).start()
            if c < H:
                s3y(c).start()

        for j in range(H):
            s3y(j).wait_recv()
            out_dma(recv3_ref, j, qd0, o3_sems.at[j]).start()
        for j in range(H):
            s3z(j).wait_recv()
            out_dma(recv3_ref, H + j, qd0, o3_sems.at[H + j]).start()

        for c in range(C):
            s1(c).wait_send()
            s2y(c).wait_send()
            s2z(c).wait_send()
        for j in range(H):
            s3y(j).wait_send()
            s3z(j).wait_send()
        for c in range(C):
            out_dma(recv1_ref, c, q0, o1_sems.at[c]).wait()
            out_dma(recv2y_ref, c, qy0, o2y_sems.at[c]).wait()
            out_dma(recv2z_ref, c, qz0, o2z_sems.at[c]).wait()
            out_dma(recv3_ref, c, qd0, o3_sems.at[c]).wait()

    return pl.pallas_call(
        body,
        in_specs=[pl.BlockSpec(memory_space=pl.ANY)],
        out_specs=pl.BlockSpec(memory_space=pl.ANY),
        out_shape=jax.ShapeDtypeStruct((m, n), jnp.bfloat16),
        scratch_shapes=[
            pltpu.VMEM((2, cr, n), jnp.float32),
            pltpu.VMEM((Q, n), jnp.bfloat16),
            pltpu.VMEM((Q, n), jnp.bfloat16),
            pltpu.VMEM((Q, n), jnp.bfloat16),
            pltpu.VMEM((Q, n), jnp.bfloat16),
            pltpu.VMEM((Q, n), jnp.bfloat16),
            pltpu.SemaphoreType.DMA((C,)),
            pltpu.SemaphoreType.DMA((C,)),
            pltpu.SemaphoreType.DMA((C,)),
            pltpu.SemaphoreType.DMA((C,)),
            pltpu.SemaphoreType.DMA((C,)),
            pltpu.SemaphoreType.DMA((C,)),
            pltpu.SemaphoreType.DMA((C,)),
            pltpu.SemaphoreType.DMA((H,)),
            pltpu.SemaphoreType.DMA((H,)),
            pltpu.SemaphoreType.DMA((H,)),
            pltpu.SemaphoreType.DMA((H,)),
            pltpu.SemaphoreType.DMA((C,)),
            pltpu.SemaphoreType.DMA((C,)),
            pltpu.SemaphoreType.DMA((C,)),
            pltpu.SemaphoreType.DMA((C,)),
        ],
        compiler_params=pltpu.CompilerParams(
            collective_id=0, vmem_limit_bytes=62 * 1024 * 1024
        ),
    )(x)


# device time: 176995 ns/iter; 2.4845x vs baseline; 1.0180x over previous
import jax
import jax.numpy as jnp
from jax import lax
from jax.experimental import pallas as pl
from jax.experimental.pallas import tpu as pltpu

C = 16
H = C // 2


def kernel(x):
    m, n = x.shape
    Q = m // 4
    cr = Q // C

    def body(
        x_ref,
        out_ref,
        stage_ref,
        xq_ref,
        recv1_ref,
        recv2y_ref,
        recv2z_ref,
        recv3_ref,
        in_sems,
        s1_send,
        s1_recv,
        s2y_send,
        s2y_recv,
        s2z_send,
        s2z_recv,
        s3y_send,
        s3y_recv,
        s3z_send,
        s3z_recv,
        o1_sems,
        o2y_sems,
        o2z_sems,
        o3_sems,
    ):
        my_x = lax.axis_index("x")
        my_y = lax.axis_index("y")
        my_z = lax.axis_index("z")
        xn = (1 - my_x, my_y, my_z)
        yn = (my_x, 1 - my_y, my_z)
        zn = (my_x, my_y, 1 - my_z)

        barrier_sem = pltpu.get_barrier_semaphore()
        for nbr in (xn, yn, zn):
            pl.semaphore_signal(
                barrier_sem,
                inc=1,
                device_id=nbr,
                device_id_type=pl.DeviceIdType.MESH,
            )
        pl.semaphore_wait(barrier_sem, 3)

        q0 = (2 * my_y + my_z) * Q
        qy0 = (2 * (1 - my_y) + my_z) * Q
        qz0 = (2 * my_y + (1 - my_z)) * Q
        qd0 = (2 * (1 - my_y) + (1 - my_z)) * Q

        def in_dma(c):
            return pltpu.make_async_copy(
                x_ref.at[pl.ds(q0 + c * cr, cr), :],
                stage_ref.at[c % 2],
                in_sems.at[c],
            )

        def rdma(src, dst, ssem, rsem, dev):
            return pltpu.make_async_remote_copy(
                src_ref=src,
                dst_ref=dst,
                send_sem=ssem,
                recv_sem=rsem,
                device_id=dev,
                device_id_type=pl.DeviceIdType.MESH,
            )

        def chunk(ref, c):
            return ref.at[pl.ds(c * cr, cr), :]

        def s1(c):
            return rdma(
                chunk(xq_ref, c), chunk(recv1_ref, c),
                s1_send.at[c], s1_recv.at[c], xn,
            )

        def s2y(c):
            return rdma(
                chunk(recv1_ref, c), chunk(recv2y_ref, c),
                s2y_send.at[c], s2y_recv.at[c], yn,
            )

        def s2z(c):
            return rdma(
                chunk(recv1_ref, c), chunk(recv2z_ref, c),
                s2z_send.at[c], s2z_recv.at[c], zn,
            )

        def s3y(j):
            return rdma(
                chunk(recv2z_ref, j), chunk(recv3_ref, j),
                s3y_send.at[j], s3y_recv.at[j], yn,
            )

        def s3z(j):
            return rdma(
                chunk(recv2y_ref, H + j), chunk(recv3_ref, H + j),
                s3z_send.at[j], s3z_recv.at[j], zn,
            )

        def out_dma(src_ref, c, row0, sem):
            return pltpu.make_async_copy(
                chunk(src_ref, c),
                out_ref.at[pl.ds(row0 + c * cr, cr), :],
                sem,
            )

        in_dma(0).start()
        if C > 1:
            in_dma(1).start()
        for c in range(C):
            in_dma(c).wait()
            xq_ref[pl.ds(c * cr, cr), :] = stage_ref[c % 2].astype(jnp.bfloat16)
            if c + 2 < C:
                in_dma(c + 2).start()
            s1(c).start()

        for c in range(C):
            s1(c).wait_recv()
            sl = pl.ds(c * cr, cr)
            recv1_ref[sl, :] = xq_ref[sl, :] + recv1_ref[sl, :]
            s2y(c).start()
            s2z(c).start()
            out_dma(recv1_ref, c, q0, o1_sems.at[c]).start()

        for c in range(C):
            s2y(c).wait_recv()
            out_dma(recv2y_ref, c, qy0, o2y_sems.at[c]).start()
            if c >= H:
                s3z(c - H).start()
            s2z(c).wait_recv()
            out_dma(recv2z_ref, c, qz0, o2z_sems.at[c]).start()
            if c < H:
                s3y(c).start()

        for j in range(H):
            s3y(j).wait_recv()
            out_dma(recv3_ref, j, qd0, o3_sems.at[j]).start()
        for j in range(H):
            s3z(j).wait_recv()
            out_dma(recv3_ref, H + j, qd0, o3_sems.at[H + j]).start()

        for c in range(C):
            s1(c).wait_send()
            s2y(c).wait_send()
            s2z(c).wait_send()
        for j in range(H):
            s3y(j).wait_send()
            s3z(j).wait_send()
        for c in range(C):
            out_dma(recv1_ref, c, q0, o1_sems.at[c]).wait()
            out_dma(recv2y_ref, c, qy0, o2y_sems.at[c]).wait()
            out_dma(recv2z_ref, c, qz0, o2z_sems.at[c]).wait()
            out_dma(recv3_ref, c, qd0, o3_sems.at[c]).wait()

    return pl.pallas_call(
        body,
        in_specs=[pl.BlockSpec(memory_space=pl.ANY)],
        out_specs=pl.BlockSpec(memory_space=pl.ANY),
        out_shape=jax.ShapeDtypeStruct((m, n), jnp.bfloat16),
        scratch_shapes=[
            pltpu.VMEM((2, cr, n), jnp.float32),
            pltpu.VMEM((Q, n), jnp.bfloat16),
            pltpu.VMEM((Q, n), jnp.bfloat16),
            pltpu.VMEM((Q, n), jnp.bfloat16),
            pltpu.VMEM((Q, n), jnp.bfloat16),
            pltpu.VMEM((Q, n), jnp.bfloat16),
            pltpu.SemaphoreType.DMA((C,)),
            pltpu.SemaphoreType.DMA((C,)),
            pltpu.SemaphoreType.DMA((C,)),
            pltpu.SemaphoreType.DMA((C,)),
            pltpu.SemaphoreType.DMA((C,)),
            pltpu.SemaphoreType.DMA((C,)),
            pltpu.SemaphoreType.DMA((C,)),
            pltpu.SemaphoreType.DMA((H,)),
            pltpu.SemaphoreType.DMA((H,)),
            pltpu.SemaphoreType.DMA((H,)),
            pltpu.SemaphoreType.DMA((H,)),
            pltpu.SemaphoreType.DMA((C,)),
            pltpu.SemaphoreType.DMA((C,)),
            pltpu.SemaphoreType.DMA((C,)),
            pltpu.SemaphoreType.DMA((C,)),
        ],
        compiler_params=pltpu.CompilerParams(
            collective_id=0, vmem_limit_bytes=62 * 1024 * 1024
        ),
    )(x)
